# Initial kernel scaffold; baseline (speedup 1.0000x reference)
import jax
import jax.numpy as jnp
from jax.experimental import pallas as pl


def kernel(x, table, pos):
    raise NotImplementedError("write your pallas kernel here")


# speedup vs baseline: 1.1253x; 1.1253x over previous
import functools

import jax
import jax.numpy as jnp
from jax import lax
from jax.experimental import pallas as pl
from jax.experimental.pallas import tpu as pltpu
from jax.experimental.pallas import tpu_sc as plsc

NC = 2
NS = 16
NW = NC * NS
LANES = 16


@functools.cache
def _make(N, V, D, L, CH):
    n_per_w = N // NW
    n_chunks = n_per_w // CH
    mesh = plsc.VectorSubcoreMesh(core_axis_name="c", subcore_axis_name="s")

    @functools.partial(
        pl.kernel,
        out_type=jax.ShapeDtypeStruct((N, D), jnp.float32),
        mesh=mesh,
        scratch_types=[
            pltpu.VMEM((n_per_w,), jnp.int32),
            pltpu.VMEM((CH, D), jnp.float32),
            pltpu.VMEM((L, D), jnp.float32),
            pltpu.SemaphoreType.DMA,
        ],
        compiler_params=pltpu.CompilerParams(use_tc_tiling_on_sc=False),
    )
    def k(x_hbm, table_hbm, pos_hbm, out_hbm, idx_v, rows_v, pos_v, sem):
        wid = lax.axis_index("s") * NC + lax.axis_index("c")
        base = wid * n_per_w
        pltpu.sync_copy(x_hbm.at[pl.ds(base, n_per_w)], idx_v)
        pltpu.sync_copy(pos_hbm.at[pl.ds(0, L)], pos_v)

        def chunk_body(c, carry):
            off = c * CH
            pltpu.async_copy(
                table_hbm.at[idx_v.at[pl.ds(off, CH)]], rows_v, sem
            ).wait()

            def row_body(r, carry2):
                for h in range(D // LANES):
                    s = h * LANES
                    rows_v[r, pl.ds(s, LANES)] = (
                        rows_v[r, pl.ds(s, LANES)] + pos_v[r, pl.ds(s, LANES)]
                    )
                return carry2

            lax.fori_loop(0, CH, row_body, 0, unroll=4)
            pltpu.sync_copy(rows_v, out_hbm.at[pl.ds(base + off, CH)])
            return carry

        lax.fori_loop(0, n_chunks, chunk_body, 0)

    return k


def kernel(x, table, pos):
    B, L = x.shape
    V, D = table.shape
    N = B * L
    out = _make(N, V, D, L, L)(x.reshape(N), table, pos[:L])
    return out.reshape(B, L, D)


# speedup vs baseline: 1.4895x; 1.3237x over previous
import functools

import jax
import jax.numpy as jnp
from jax import lax
from jax.experimental import pallas as pl
from jax.experimental.pallas import tpu as pltpu
from jax.experimental.pallas import tpu_sc as plsc

NC = 2
NS = 16
NW = NC * NS
LANES = 16
NBUF = 3


@functools.cache
def _make(N, V, D, L, CH):
    n_per_w = N // NW
    n_chunks = n_per_w // CH
    reps = CH // L
    mesh = plsc.VectorSubcoreMesh(core_axis_name="c", subcore_axis_name="s")

    @functools.partial(
        pl.kernel,
        out_type=jax.ShapeDtypeStruct((N, D), jnp.float32),
        mesh=mesh,
        scratch_types=[
            pltpu.VMEM((n_per_w,), jnp.int32),
            pltpu.VMEM((L, D), jnp.float32),
        ]
        + [pltpu.VMEM((CH, D), jnp.float32)] * NBUF
        + [pltpu.SemaphoreType.DMA] * (2 * NBUF),
        compiler_params=pltpu.CompilerParams(use_tc_tiling_on_sc=False),
    )
    def k(x_hbm, table_hbm, pos_hbm, out_hbm, idx_v, pos_v, *bufs_and_sems):
        bufs = bufs_and_sems[:NBUF]
        gsems = bufs_and_sems[NBUF:2 * NBUF]
        ssems = bufs_and_sems[2 * NBUF:3 * NBUF]

        wid = lax.axis_index("s") * NC + lax.axis_index("c")
        base = wid * n_per_w
        pltpu.sync_copy(x_hbm.at[pl.ds(base, n_per_w)], idx_v)
        pltpu.sync_copy(pos_hbm.at[pl.ds(0, L)], pos_v)

        def start_gather(c):
            b = c % NBUF
            return pltpu.async_copy(
                table_hbm.at[idx_v.at[pl.ds(c * CH, CH)]], bufs[b], gsems[b]
            )

        def add_pos(b):
            rows_v = bufs[b]

            def row_body(r, carry):
                for h in range(D // LANES):
                    s = h * LANES
                    pv = pos_v[r, pl.ds(s, LANES)]
                    for sb in range(reps):
                        rows_v[sb * L + r, pl.ds(s, LANES)] = (
                            rows_v[sb * L + r, pl.ds(s, LANES)] + pv
                        )
                return carry

            lax.fori_loop(0, L, row_body, 0, unroll=4)

        def start_store(c):
            b = c % NBUF
            return pltpu.async_copy(
                bufs[b], out_hbm.at[pl.ds(base + c * CH, CH)], ssems[b]
            )

        gathers = {}
        stores = {}
        for c in range(min(2, n_chunks)):
            gathers[c] = start_gather(c)
        for c in range(n_chunks):
            b = c % NBUF
            gathers.pop(c).wait()
            add_pos(b)
            if c + 2 < n_chunks:
                if c - 1 >= 0:
                    stores.pop(c - 1).wait()
                gathers[c + 2] = start_gather(c + 2)
            stores[c] = start_store(c)
        for c in sorted(stores):
            stores.pop(c).wait()

    return k


def kernel(x, table, pos):
    B, L = x.shape
    V, D = table.shape
    N = B * L
    out = _make(N, V, D, L, 4 * L)(x.reshape(N), table, pos[:L])
    return out.reshape(B, L, D)
